# Initial kernel scaffold; baseline (speedup 1.0000x reference)
import jax
import jax.numpy as jnp
from jax.experimental import pallas as pl


def kernel(inputs, embeddings):
    raise NotImplementedError("write your pallas kernel here")


# speedup vs baseline: 12.8542x; 12.8542x over previous
import functools

import jax
import jax.numpy as jnp
from jax import lax
from jax.experimental import pallas as pl
from jax.experimental.pallas import tpu as pltpu
from jax.experimental.pallas import tpu_sc as plsc

V = 8192
D = 32
N = 32 * 1024
R = 256
NB = N // R

_LOSS_SCALE = 2.0 / float(N * D)


_H = V // 2


def _tc_body(x_ref, e_ref, a_ref, b_ref, idx_ref, loss_ref):
    i = pl.program_id(0)
    x = x_ref[...]
    e = e_ref[...]
    a = a_ref[...]
    b = b_ref[...]
    c = lax.dot_general(x, e, (((1,), (1,)), ((), ())),
                        preferred_element_type=jnp.float32)
    d = (a + b) - 2.0 * c
    d1 = d[:, :_H]
    d2 = d[:, _H:]
    m1 = jnp.min(d1, axis=1, keepdims=True)
    m2 = jnp.min(d2, axis=1, keepdims=True)
    iota = lax.broadcasted_iota(jnp.int32, (R, _H), 1)
    idx1 = jnp.min(jnp.where(d1 == m1, iota, V), axis=1, keepdims=True)
    idx2 = jnp.min(jnp.where(d2 == m2, iota, V), axis=1, keepdims=True) + _H
    m1b = m1.astype(jnp.bfloat16).astype(jnp.float32)
    take2 = m2 < m1b
    idx = jnp.where(take2, idx2, idx1)
    idx_ref[...] = idx
    chosen = jnp.where(take2, m2, m1)
    part = jnp.sum(chosen).reshape(1, 1)
    prev = jnp.where(i == 0, jnp.zeros_like(part), loss_ref[...])
    acc = prev + part
    scale = jnp.where(i == NB - 1, jnp.float32(_LOSS_SCALE), jnp.float32(1.0))
    loss_ref[...] = acc * scale


def _tc_argmin(x, e, a, b):
    return pl.pallas_call(
        _tc_body,
        grid=(NB,),
        in_specs=[
            pl.BlockSpec((R, D), lambda i: (i, 0)),
            pl.BlockSpec((V, D), lambda i: (0, 0)),
            pl.BlockSpec((R, 1), lambda i: (i, 0)),
            pl.BlockSpec((1, V), lambda i: (0, 0)),
        ],
        out_specs=[
            pl.BlockSpec((R, 1), lambda i: (i, 0)),
            pl.BlockSpec((1, 1), lambda i: (0, 0)),
        ],
        out_shape=[
            jax.ShapeDtypeStruct((N, 1), jnp.int32),
            jax.ShapeDtypeStruct((1, 1), jnp.float32),
        ],
        compiler_params=pltpu.CompilerParams(
            dimension_semantics=("arbitrary",)),
    )(x, e, a, b)


_NC = 2
_NS = 16
_NW = _NC * _NS
_BPW = N // _NW

@functools.lru_cache(maxsize=1)
def _make_sc_gather():
    mesh = plsc.VectorSubcoreMesh(
        core_axis_name="c", subcore_axis_name="s",
        num_cores=_NC, num_subcores=_NS)

    @functools.partial(
        pl.kernel,
        out_type=jax.ShapeDtypeStruct((N, D), jnp.float32),
        mesh=mesh,
        scratch_types=[
            pltpu.VMEM((_BPW,), jnp.int32),
            pltpu.VMEM((_BPW, D), jnp.float32),
            pltpu.SemaphoreType.DMA,
        ],
        compiler_params=pltpu.CompilerParams(use_tc_tiling_on_sc=False),
    )
    def _sc_gather(idx_hbm, table_hbm, out_hbm, idx_v, rows_v, sem):
        wid = lax.axis_index("s") * _NC + lax.axis_index("c")
        base = wid * _BPW
        pltpu.sync_copy(idx_hbm.at[pl.ds(base, _BPW)], idx_v)
        pltpu.async_copy(table_hbm.at[idx_v], rows_v, sem).wait()
        pltpu.sync_copy(rows_v, out_hbm.at[pl.ds(base, _BPW)])

    return _sc_gather


def kernel(inputs, embeddings):
    x = inputs.reshape(N, D)
    a = jnp.sum(x ** 2, axis=1, keepdims=True)
    b = jnp.sum(embeddings ** 2, axis=1)[None, :]
    idx2, loss2 = _tc_argmin(x, embeddings, a, b)
    idx_flat = idx2.reshape(N)
    q = _make_sc_gather()(idx_flat, embeddings)
    quantized_st = q.reshape(inputs.shape)
    loss = loss2[0, 0]
    indices = idx_flat.reshape(inputs.shape[0], inputs.shape[1])
    return quantized_st, loss, indices
